# baseline (speedup 1.0000x reference)
import functools

import jax
import jax.numpy as jnp
from jax import lax
from jax.experimental import pallas as pl
from jax.experimental.pallas import tpu as pltpu
from jax.experimental.pallas import tpu_sc as plsc

B = 128
MAX_N = 32768
SET_CH = 256
K = 1024

_NC = 2
_NS = 16
_NW = _NC * _NS
_R = B * K
_RPW = _R // _NW
_C = 128
_NCH = _RPW // _C


def _dot_nt(a, b):
    return jax.lax.dot_general(a, b, (((1,), (1,)), ((), ())),
                               preferred_element_type=jnp.float32)


def _dense_body(ang_ref, ap_ref, cos_ref):
    cos_ref[...] = _dot_nt(ang_ref[...], ap_ref[...])


def _dense(angles, angles_params):
    return pl.pallas_call(
        _dense_body,
        out_shape=jax.ShapeDtypeStruct((B, MAX_N), jnp.float32),
    )(angles, angles_params)


def _film_body(idx_hbm, s_hbm, points_hbm, w1_hbm, b1_hbm, w2_hbm, b2_hbm,
               out_hbm, idx_v, s_v, rows_v, out_v, w1_v, b1_v, w2_v, b2_v, sem):
    wid = lax.axis_index("s") * _NC + lax.axis_index("c")
    base = wid * _RPW
    pltpu.sync_copy(idx_hbm.at[pl.ds(base, _RPW)], idx_v)
    pltpu.sync_copy(s_hbm.at[pl.ds(base, _RPW)], s_v)
    pltpu.sync_copy(w1_hbm, w1_v)
    pltpu.sync_copy(b1_hbm, b1_v)
    pltpu.sync_copy(w2_hbm, w2_v)
    pltpu.sync_copy(b2_hbm, b2_v)

    def chunk_body(j, carry):
        cb = j * _C
        pltpu.async_copy(points_hbm.at[idx_v.at[pl.ds(cb, _C)]], rows_v, sem).wait()

        def row_body(r16, carry2):
            s16 = s_v[pl.ds(cb + r16 * 16, 16)]
            for j in range(16):
                r = r16 * 16 + j
                a16 = jnp.full((16,), s16[j])
                for c in range(SET_CH // 16):
                    sl = pl.ds(c * 16, 16)
                    p = rows_v[r, sl]
                    al = a16 * w1_v[sl] + b1_v[sl]
                    be = a16 * w2_v[sl] + b2_v[sl]
                    out_v[r, sl] = al * p + be
            return carry2

        lax.fori_loop(0, _C // 16, row_body, 0, unroll=False)
        pltpu.sync_copy(out_v, out_hbm.at[pl.ds(base + cb, _C)])
        return carry

    lax.fori_loop(0, _NCH, chunk_body, 0, unroll=False)


@functools.partial(jax.jit, static_argnames=())
def _film(idx_flat, s_flat, points, w1, b1, w2, b2):
    kern = pl.kernel(
        _film_body,
        mesh=plsc.VectorSubcoreMesh(core_axis_name="c", subcore_axis_name="s"),
        out_type=jax.ShapeDtypeStruct((_R, SET_CH), jnp.float32),
        scratch_types=[
            pltpu.VMEM((_RPW,), jnp.int32),
            pltpu.VMEM((_RPW,), jnp.float32),
            pltpu.VMEM((_C, SET_CH), jnp.float32),
            pltpu.VMEM((_C, SET_CH), jnp.float32),
            pltpu.VMEM((SET_CH,), jnp.float32),
            pltpu.VMEM((SET_CH,), jnp.float32),
            pltpu.VMEM((SET_CH,), jnp.float32),
            pltpu.VMEM((SET_CH,), jnp.float32),
            pltpu.SemaphoreType.DMA,
        ],
    )
    return kern(idx_flat, s_flat, points, w1, b1, w2, b2)


def kernel(latent, points, angles_params, mlp_w1, mlp_b1, mlp_w2, mlp_b2,
           lin1_w, lin1_b, lin2_w, lin2_b, n):
    h = jax.nn.relu(latent @ mlp_w1 + mlp_b1)
    angles = h @ mlp_w2 + mlp_b2
    angles = angles / (jnp.linalg.norm(angles, axis=1)[:, None] + 1e-05)
    cos = _dense(angles, angles_params)
    probs = jax.nn.softmax(cos, axis=1)
    srted, indices = lax.top_k(probs, K)
    s_flat = srted.reshape(-1)
    idx_flat = indices.reshape(-1).astype(jnp.int32)
    nf = jnp.asarray(n, jnp.float32)
    w1 = nf * lin1_w.reshape(-1)
    w2 = nf * lin2_w.reshape(-1)
    out = _film(idx_flat, s_flat, points, w1, lin1_b, w2, lin2_b)
    return out.reshape(B, K, SET_CH)


# speedup vs baseline: 1.2697x; 1.2697x over previous
import functools

import jax
import jax.numpy as jnp
from jax import lax
from jax.experimental import pallas as pl
from jax.experimental.pallas import tpu as pltpu
from jax.experimental.pallas import tpu_sc as plsc

B = 128
MAX_N = 32768
SET_CH = 256
K = 1024

_NC = 2
_NS = 16
_NW = _NC * _NS
_R = B * K
_RPW = _R // _NW
_C = 128
_NCH = _RPW // _C


def _dot_nt(a, b):
    return jax.lax.dot_general(a, b, (((1,), (1,)), ((), ())),
                               preferred_element_type=jnp.float32)


def _dense_body(ang_ref, ap_ref, cos_ref):
    cos_ref[...] = _dot_nt(ang_ref[...], ap_ref[...])


def _dense(angles, angles_params):
    return pl.pallas_call(
        _dense_body,
        out_shape=jax.ShapeDtypeStruct((B, MAX_N), jnp.float32),
    )(angles, angles_params)


def _film_body(idx_hbm, s_hbm, points_hbm, w1_hbm, b1_hbm, w2_hbm, b2_hbm,
               out_hbm, idx_v, s_v, rows0, rows1, a_buf, w1_v, b1_v, w2_v, b2_v,
               sg0, sg1, ss0, ss1):
    wid = lax.axis_index("s") * _NC + lax.axis_index("c")
    base = wid * _RPW
    pltpu.sync_copy(idx_hbm.at[pl.ds(base, _RPW)], idx_v)
    pltpu.sync_copy(s_hbm.at[pl.ds(base, _RPW)], s_v)
    pltpu.sync_copy(w1_hbm, w1_v)
    pltpu.sync_copy(b1_hbm, b1_v)
    pltpu.sync_copy(w2_hbm, w2_v)
    pltpu.sync_copy(b2_hbm, b2_v)

    rows = (rows0, rows1)
    sg = (sg0, sg1)
    ss = (ss0, ss1)

    def compute_chunk(j, buf):
        cb = j * _C
        for r16 in range(_C // 16):
            s16 = s_v[pl.ds(cb + r16 * 16, 16)]
            for t in range(16):
                a_buf[r16 * 16 + t, :] = jnp.full((16,), s16[t])
        for c in range(SET_CH // 16):
            sl = pl.ds(c * 16, 16)
            w1c = w1_v[sl]
            b1c = b1_v[sl]
            w2c = w2_v[sl]
            b2c = b2_v[sl]

            def rbody(r, carry2):
                a16 = a_buf[r, pl.ds(0, 16)]
                p = buf[r, sl]
                buf[r, sl] = (a16 * w1c + b1c) * p + (a16 * w2c + b2c)
                return carry2

            lax.fori_loop(0, _C, rbody, 0, unroll=8)

    pltpu.async_copy(points_hbm.at[idx_v.at[pl.ds(0, _C)]], rows[0], sg[0])

    def pair_body(i, carry):
        for par in (0, 1):
            j = 2 * i + par
            buf = rows[par]
            other = rows[1 - par]
            pltpu.make_async_copy(
                points_hbm.at[idx_v.at[pl.ds(j * _C, _C)]], buf, sg[par]).wait()

            @pl.when(j > 0)
            def _():
                pltpu.make_async_copy(
                    other, out_hbm.at[pl.ds(base + (j - 1) * _C, _C)],
                    ss[1 - par]).wait()

            @pl.when(j < _NCH - 1)
            def _():
                pltpu.async_copy(
                    points_hbm.at[idx_v.at[pl.ds((j + 1) * _C, _C)]], other,
                    sg[1 - par])

            compute_chunk(j, buf)
            pltpu.async_copy(buf, out_hbm.at[pl.ds(base + j * _C, _C)], ss[par])
        return carry

    lax.fori_loop(0, _NCH // 2, pair_body, 0, unroll=False)
    pltpu.make_async_copy(
        rows[1], out_hbm.at[pl.ds(base + (_NCH - 1) * _C, _C)], ss[1]).wait()


@functools.partial(jax.jit, static_argnames=())
def _film(idx_flat, s_flat, points, w1, b1, w2, b2):
    kern = pl.kernel(
        _film_body,
        mesh=plsc.VectorSubcoreMesh(core_axis_name="c", subcore_axis_name="s"),
        out_type=jax.ShapeDtypeStruct((_R, SET_CH), jnp.float32),
        scratch_types=[
            pltpu.VMEM((_RPW,), jnp.int32),
            pltpu.VMEM((_RPW,), jnp.float32),
            pltpu.VMEM((_C, SET_CH), jnp.float32),
            pltpu.VMEM((_C, SET_CH), jnp.float32),
            pltpu.VMEM((_C, 16), jnp.float32),
            pltpu.VMEM((SET_CH,), jnp.float32),
            pltpu.VMEM((SET_CH,), jnp.float32),
            pltpu.VMEM((SET_CH,), jnp.float32),
            pltpu.VMEM((SET_CH,), jnp.float32),
            pltpu.SemaphoreType.DMA,
            pltpu.SemaphoreType.DMA,
            pltpu.SemaphoreType.DMA,
            pltpu.SemaphoreType.DMA,
        ],
    )
    return kern(idx_flat, s_flat, points, w1, b1, w2, b2)


def kernel(latent, points, angles_params, mlp_w1, mlp_b1, mlp_w2, mlp_b2,
           lin1_w, lin1_b, lin2_w, lin2_b, n):
    h = jax.nn.relu(latent @ mlp_w1 + mlp_b1)
    angles = h @ mlp_w2 + mlp_b2
    angles = angles / (jnp.linalg.norm(angles, axis=1)[:, None] + 1e-05)
    cos = _dense(angles, angles_params)
    probs = jax.nn.softmax(cos, axis=1)
    srted, indices = lax.top_k(probs, K)
    s_flat = srted.reshape(-1)
    idx_flat = indices.reshape(-1).astype(jnp.int32)
    nf = jnp.asarray(n, jnp.float32)
    w1 = nf * lin1_w.reshape(-1)
    w2 = nf * lin2_w.reshape(-1)
    out = _film(idx_flat, s_flat, points, w1, lin1_b, w2, lin2_b)
    return out.reshape(B, K, SET_CH)
